# Initial kernel scaffold; baseline (speedup 1.0000x reference)
import jax
import jax.numpy as jnp
from jax.experimental import pallas as pl


def kernel(adj_set, adj_static):
    raise NotImplementedError("write your pallas kernel here")


# speedup vs baseline: 5.1394x; 5.1394x over previous
import functools

import jax
import jax.numpy as jnp
from jax.experimental import pallas as pl
from jax.experimental.pallas import tpu as pltpu


def _body(a_ref, s0_ref, out_static_ref, out_set_ref, cum_ref):
    t = pl.program_id(1)

    @pl.when(t == 0)
    def _init():
        cum_ref[...] = jnp.ones_like(cum_ref)

    a = a_ref[0, 0]
    s0 = s0_ref[0]
    cum = cum_ref[...]

    g = s0 * cum
    gt = jnp.swapaxes(g, 1, 2)
    n = g.shape[-1]
    row = jax.lax.broadcasted_iota(jnp.int32, (n, n), 0)
    col = jax.lax.broadcasted_iota(jnp.int32, (n, n), 1)
    eye = jnp.where(row == col, jnp.float32(1.0), jnp.float32(0.0))
    adj_sum = g + gt + eye[None]
    mask_d = jnp.where(adj_sum > 0, jnp.float32(1.0), jnp.float32(0.0))
    out_set_ref[0, 0] = a * mask_d

    maxa = jnp.max(a, axis=0, keepdims=True)
    keep = jnp.where(a <= maxa, jnp.float32(1.0), jnp.float32(0.0))
    cum = cum * keep
    cum_ref[...] = cum
    out_static_ref[0, 0] = s0 * cum


@functools.partial(jax.jit, static_argnames=())
def kernel(adj_set, adj_static):
    T, B, H, N, _ = adj_set.shape
    out_shape = jax.ShapeDtypeStruct((T, B, H, N, N), adj_set.dtype)
    grid = (B, T)
    out_static, out_set = pl.pallas_call(
        _body,
        grid=grid,
        in_specs=[
            pl.BlockSpec((1, 1, H, N, N), lambda b, t: (t, b, 0, 0, 0)),
            pl.BlockSpec((1, H, N, N), lambda b, t: (b, 0, 0, 0)),
        ],
        out_specs=[
            pl.BlockSpec((1, 1, H, N, N), lambda b, t: (t, b, 0, 0, 0)),
            pl.BlockSpec((1, 1, H, N, N), lambda b, t: (t, b, 0, 0, 0)),
        ],
        out_shape=[out_shape, out_shape],
        scratch_shapes=[pltpu.VMEM((H, N, N), jnp.float32)],
        compiler_params=pltpu.CompilerParams(
            dimension_semantics=("arbitrary", "arbitrary"),
        ),
    )(adj_set, adj_static)
    return out_static, out_set
